# Initial kernel scaffold; baseline (speedup 1.0000x reference)
import jax
import jax.numpy as jnp
from jax.experimental import pallas as pl


def kernel(boxes_sml0, i_feat, y_feat, W_vs, b_vs, W_ts, b_ts):
    raise NotImplementedError("write your pallas kernel here")


# speedup vs baseline: 1.1735x; 1.1735x over previous
import functools

import jax
import jax.numpy as jnp
from jax import lax
from jax.experimental import pallas as pl
from jax.experimental.pallas import tpu as pltpu

BS = 8
GRID = 6400
ANN = 3
CH = 85
ROW = ANN * CH
SEL = 100
HID = 512
VIS = 1024
GB = 256
NBLK = GRID // GB
SUBL = 50
LANE = 128


def _obj_kernel(box_ref, out_ref):
    x = box_ref[...]
    col = lax.broadcasted_iota(jnp.int32, (1, 1, ROW), 2)
    mask = ((col == 4) | (col == 4 + CH) | (col == 4 + 2 * CH)).astype(jnp.float32)
    out_ref[...] = jnp.sum(x * mask, axis=2)


def _objectness(boxes3):
    return pl.pallas_call(
        _obj_kernel,
        grid=(NBLK,),
        in_specs=[pl.BlockSpec((BS, GB, ROW), lambda i: (0, i, 0))],
        out_specs=pl.BlockSpec((BS, GB), lambda i: (0, i)),
        out_shape=jax.ShapeDtypeStruct((BS, GRID), jnp.float32),
    )(boxes3)


def _select_kernel(obj_ref, selg_ref):
    obj = obj_ref[...]
    bits = lax.bitcast_convert_type(obj, jnp.int32)
    s = jnp.where(bits >= 0, bits, bits ^ jnp.int32(0x7FFFFFFF))

    def body(i, cur):
        bit = 31 - i
        cand = cur + lax.shift_left(jnp.int32(1), bit)
        cnt = jnp.sum((s >= cand).astype(jnp.int32), axis=(1, 2), keepdims=True)
        return jnp.where(cnt >= SEL, cand, cur)

    t = lax.fori_loop(0, 32, body, jnp.full((BS, 1, 1), jnp.iinfo(jnp.int32).min,
                                            dtype=jnp.int32))

    iu = lax.broadcasted_iota(jnp.int32, (LANE, LANE), 0)
    ju = lax.broadcasted_iota(jnp.int32, (LANE, LANE), 1)
    U128 = (iu <= ju).astype(jnp.float32)
    I128 = (iu == ju).astype(jnp.float32)
    ir = lax.broadcasted_iota(jnp.int32, (SUBL, SUBL), 0)
    jr = lax.broadcasted_iota(jnp.int32, (SUBL, SUBL), 1)
    Lstrict = (jr < ir).astype(jnp.float32)
    I50 = (ir == jr).astype(jnp.float32)
    ones_1x50 = jnp.ones((1, SUBL), jnp.float32)
    ones_1x128 = jnp.ones((1, LANE), jnp.float32)
    iota50c = lax.broadcasted_iota(jnp.int32, (SUBL, 1), 0).astype(jnp.float32)
    iota128c = lax.broadcasted_iota(jnp.int32, (LANE, 1), 0).astype(jnp.float32)
    kp1 = iota128c + 1.0

    for b in range(BS):
        sb = s[b]
        tb = t[b]
        m_gt = (sb > tb).astype(jnp.float32)
        m_eq = (sb == tb).astype(jnp.float32)
        n_gt = jnp.sum(m_gt)
        quota = jnp.float32(SEL) - n_gt

        r1e = jnp.dot(m_eq, U128, preferred_element_type=jnp.float32)
        rse = jnp.sum(m_eq, axis=1, keepdims=True)
        starts_e = jnp.dot(Lstrict, rse, preferred_element_type=jnp.float32)
        sel_m = jnp.logical_or(m_gt > 0.5,
                               jnp.logical_and(m_eq > 0.5,
                                               r1e + starts_e <= quota))
        self_f = sel_m.astype(jnp.float32)

        r1s = jnp.dot(self_f, U128, preferred_element_type=jnp.float32)
        rs = jnp.sum(self_f, axis=1, keepdims=True)
        starts = jnp.dot(Lstrict, rs, preferred_element_type=jnp.float32)
        ends = starts + rs
        starts_r = jnp.dot(ones_1x50, I50 * starts, preferred_element_type=jnp.float32)
        ends_r = jnp.dot(ones_1x50, I50 * ends, preferred_element_type=jnp.float32)
        ohr = jnp.logical_and(kp1 > starts_r, kp1 <= ends_r).astype(jnp.float32)
        rows_rank = jnp.dot(ohr, r1s, preferred_element_type=jnp.float32)
        rows_sel = jnp.dot(ohr, self_f, preferred_element_type=jnp.float32)
        starts_k = jnp.dot(ohr, starts, preferred_element_type=jnp.float32)
        tgt = kp1 - starts_k
        m2 = jnp.logical_and(rows_rank == tgt, rows_sel > 0.5).astype(jnp.float32)
        c_k = jnp.dot(m2, iota128c, preferred_element_type=jnp.float32)
        r_k = jnp.dot(ohr, iota50c, preferred_element_type=jnp.float32)
        r_row = jnp.dot(ones_1x128, I128 * r_k, preferred_element_type=jnp.float32)
        c_row = jnp.dot(ones_1x128, I128 * c_k, preferred_element_type=jnp.float32)
        g_row = r_row * jnp.float32(LANE) + c_row
        selg_ref[b : b + 1, :] = g_row.astype(jnp.int32)


def _select(obj3):
    return pl.pallas_call(
        _select_kernel,
        in_specs=[pl.BlockSpec(memory_space=pltpu.MemorySpace.VMEM)],
        out_specs=pl.BlockSpec(memory_space=pltpu.MemorySpace.VMEM),
        out_shape=jax.ShapeDtypeStruct((BS, LANE), jnp.int32),
    )(obj3)


def _dot_f32(a, b):
    a1 = a.astype(jnp.bfloat16)
    ra = a - a1.astype(jnp.float32)
    a2 = ra.astype(jnp.bfloat16)
    a3 = (ra - a2.astype(jnp.float32)).astype(jnp.bfloat16)
    b1 = b.astype(jnp.bfloat16)
    rb = b - b1.astype(jnp.float32)
    b2 = rb.astype(jnp.bfloat16)
    b3 = (rb - b2.astype(jnp.float32)).astype(jnp.bfloat16)

    def d(x, y):
        return jnp.dot(x, y, preferred_element_type=jnp.float32)

    return ((d(a3, b1) + d(a1, b3)) + (d(a2, b2) +
            (d(a2, b1) + d(a1, b2))) + d(a1, b1))


def _score_kernel(selg_smem, selg3_ref, ifeat_hbm, wvs_ref, yf_ref, wts_ref,
                  bts_ref, bvs_ref, max_ref, gb_ref, gbuf, sems):
    b = pl.program_id(0)
    copies = []
    for k in range(SEL):
        idx = selg_smem[b, k] + b * GRID
        cp = pltpu.make_async_copy(
            ifeat_hbm.at[pl.ds(idx, 1), :],
            gbuf.at[pl.ds(k, 1), :],
            sems.at[k],
        )
        cp.start()
        copies.append(cp)

    y_new = _dot_f32(yf_ref[...], wts_ref[...]) + bts_ref[...]
    row_b = (lax.broadcasted_iota(jnp.int32, (BS, 1), 0) == b)
    y_b = jnp.sum(jnp.where(row_b, y_new, 0.0), axis=0, keepdims=True)

    for cp in copies:
        cp.wait()

    x_new = _dot_f32(gbuf[...], wvs_ref[...]) + bvs_ref[...]
    scores = jnp.sum(x_new * y_b, axis=1, keepdims=True)
    kcol = lax.broadcasted_iota(jnp.int32, (LANE, 1), 0).astype(jnp.float32)
    valid = kcol < jnp.float32(SEL)
    scores = jnp.where(valid, scores, -jnp.inf)
    maxv = jnp.max(scores)
    cand = jnp.where(scores == maxv, kcol, jnp.float32(1e9))
    bestk = jnp.min(cand)
    ohk = (lax.broadcasted_iota(jnp.int32, (1, LANE), 1).astype(jnp.float32) == bestk)
    selv = selg3_ref[0].astype(jnp.float32)
    gbest = jnp.sum(jnp.where(ohk, selv, 0.0))
    max_ref[...] = jnp.full((1, 1, LANE), maxv, jnp.float32)
    gb_ref[...] = jnp.full((1, 1, LANE), gbest).astype(jnp.int32)


def _score(selg, selg3, ifeat2, w_vs, y_feat, w_ts, bts2, bvs2):
    return pl.pallas_call(
        _score_kernel,
        grid=(BS,),
        in_specs=[
            pl.BlockSpec(memory_space=pltpu.MemorySpace.SMEM),
            pl.BlockSpec((1, 1, LANE), lambda b: (b, 0, 0)),
            pl.BlockSpec(memory_space=pltpu.MemorySpace.HBM),
            pl.BlockSpec((VIS, HID), lambda b: (0, 0)),
            pl.BlockSpec((BS, HID), lambda b: (0, 0)),
            pl.BlockSpec((HID, HID), lambda b: (0, 0)),
            pl.BlockSpec((1, HID), lambda b: (0, 0)),
            pl.BlockSpec((1, HID), lambda b: (0, 0)),
        ],
        out_specs=[
            pl.BlockSpec((1, 1, LANE), lambda b: (b, 0, 0)),
            pl.BlockSpec((1, 1, LANE), lambda b: (b, 0, 0)),
        ],
        out_shape=[
            jax.ShapeDtypeStruct((BS, 1, LANE), jnp.float32),
            jax.ShapeDtypeStruct((BS, 1, LANE), jnp.int32),
        ],
        scratch_shapes=[
            pltpu.VMEM((LANE, VIS), jnp.float32),
            pltpu.SemaphoreType.DMA((SEL,)),
        ],
    )(selg, selg3, ifeat2, w_vs, y_feat, w_ts, bts2, bvs2)


def _decode_kernel(gb_ref, box_ref, out_ref):
    b = pl.program_id(0)
    sub = jnp.remainder(gb_ref[b], 8)
    ohc = lax.broadcasted_iota(jnp.int32, (8, 1), 0) == sub
    row = jnp.sum(jnp.where(ohc, box_ref[0], 0.0), axis=0, keepdims=True)

    def chan(a, j):
        return row[:, a * CH + j : a * CH + j + 1]

    xs, ys, x1s, y1s, scs = [], [], [], [], []
    for a in range(ANN):
        cx, cy, w, h, sc = (chan(a, 0), chan(a, 1), chan(a, 2), chan(a, 3),
                            chan(a, 4))
        x0 = cx - w * 0.5
        y0 = cy - h * 0.5
        xs.append(x0)
        ys.append(y0)
        x1s.append(x0 + w)
        y1s.append(y0 + h)
        scs.append(sc)

    use0 = jnp.logical_and(scs[0] >= scs[1], scs[0] >= scs[2])
    use1 = jnp.logical_and(jnp.logical_not(use0), scs[1] >= scs[2])

    def pick(v):
        return jnp.where(use0, v[0], jnp.where(use1, v[1], v[2]))

    bx0, by0, bx1, by1, bsc = pick(xs), pick(ys), pick(x1s), pick(y1s), pick(scs)
    lane = lax.broadcasted_iota(jnp.int32, (1, 1, LANE), 2)
    out = jnp.where(lane == 0, bx0,
          jnp.where(lane == 1, by0,
          jnp.where(lane == 2, bx1,
          jnp.where(lane == 3, by1, bsc))))
    out_ref[...] = out


def _decode(gb, boxes3):
    grid_spec = pltpu.PrefetchScalarGridSpec(
        num_scalar_prefetch=1,
        grid=(BS,),
        in_specs=[pl.BlockSpec((1, 8, ROW), lambda b, gb: (b, gb[b] // 8, 0))],
        out_specs=pl.BlockSpec((1, 1, LANE), lambda b, gb: (b, 0, 0)),
    )
    return pl.pallas_call(
        _decode_kernel,
        grid_spec=grid_spec,
        out_shape=jax.ShapeDtypeStruct((BS, 1, LANE), jnp.float32),
    )(gb, boxes3)


@jax.jit
def kernel(boxes_sml0, i_feat, y_feat, W_vs, b_vs, W_ts, b_ts):
    boxes3 = boxes_sml0.reshape(BS, GRID, ROW)
    obj = _objectness(boxes3)
    obj3 = obj.reshape(BS, SUBL, LANE)
    selg = _select(obj3)
    selg3 = selg.reshape(BS, 1, LANE)
    ifeat2 = i_feat.reshape(BS * GRID, VIS)
    maxpad, gbpad = _score(selg, selg3, ifeat2, W_vs, y_feat, W_ts,
                           b_ts.reshape(1, HID), b_vs.reshape(1, HID))
    maxval = maxpad[:, 0, 0]
    gb = gbpad[:, 0, 0]
    boxrow = _decode(gb, boxes3)
    box_new = boxrow[:, 0, :5].reshape(BS, 1, 5)
    return box_new, maxval


# speedup vs baseline: 1.3658x; 1.1639x over previous
import functools

import jax
import jax.numpy as jnp
from jax import lax
from jax.experimental import pallas as pl
from jax.experimental.pallas import tpu as pltpu

BS = 8
GRID = 6400
ANN = 3
CH = 85
ROW = ANN * CH
SEL = 100
HID = 512
VIS = 1024
GB = 256
NBLK = GRID // GB
SUBL = 50
LANE = 128


def _obj_kernel(box_ref, out_ref):
    x = box_ref[...]
    col = lax.broadcasted_iota(jnp.int32, (1, 1, ROW), 2)
    mask = ((col == 4) | (col == 4 + CH) | (col == 4 + 2 * CH)).astype(jnp.float32)
    out_ref[...] = jnp.sum(x * mask, axis=2)


def _objectness(boxes3):
    return pl.pallas_call(
        _obj_kernel,
        grid=(NBLK,),
        in_specs=[pl.BlockSpec((BS, GB, ROW), lambda i: (0, i, 0))],
        out_specs=pl.BlockSpec((BS, GB), lambda i: (0, i)),
        out_shape=jax.ShapeDtypeStruct((BS, GRID), jnp.float32),
    )(boxes3)


def _select_kernel(obj_ref, selg_ref):
    obj = obj_ref[...]
    bits = lax.bitcast_convert_type(obj, jnp.int32)
    s = jnp.where(bits >= 0, bits, bits ^ jnp.int32(0x7FFFFFFF))

    def body(i, cur):
        bit = 31 - i
        cand = cur + lax.shift_left(jnp.int32(1), bit)
        cnt = jnp.sum((s >= cand).astype(jnp.int32), axis=(1, 2), keepdims=True)
        return jnp.where(cnt >= SEL, cand, cur)

    t = lax.fori_loop(0, 32, body, jnp.full((BS, 1, 1), jnp.iinfo(jnp.int32).min,
                                            dtype=jnp.int32))

    iu = lax.broadcasted_iota(jnp.int32, (LANE, LANE), 0)
    ju = lax.broadcasted_iota(jnp.int32, (LANE, LANE), 1)
    U128 = (iu <= ju).astype(jnp.float32)
    I128 = (iu == ju).astype(jnp.float32)
    ir = lax.broadcasted_iota(jnp.int32, (SUBL, SUBL), 0)
    jr = lax.broadcasted_iota(jnp.int32, (SUBL, SUBL), 1)
    Lstrict = (jr < ir).astype(jnp.float32)
    I50 = (ir == jr).astype(jnp.float32)
    ones_1x50 = jnp.ones((1, SUBL), jnp.float32)
    ones_1x128 = jnp.ones((1, LANE), jnp.float32)
    iota50c = lax.broadcasted_iota(jnp.int32, (SUBL, 1), 0).astype(jnp.float32)
    iota128c = lax.broadcasted_iota(jnp.int32, (LANE, 1), 0).astype(jnp.float32)
    kp1 = iota128c + 1.0

    for b in range(BS):
        sb = s[b]
        tb = t[b]
        m_gt = (sb > tb).astype(jnp.float32)
        m_eq = (sb == tb).astype(jnp.float32)
        n_gt = jnp.sum(m_gt)
        quota = jnp.float32(SEL) - n_gt

        r1e = jnp.dot(m_eq, U128, preferred_element_type=jnp.float32)
        rse = jnp.sum(m_eq, axis=1, keepdims=True)
        starts_e = jnp.dot(Lstrict, rse, preferred_element_type=jnp.float32)
        sel_m = jnp.logical_or(m_gt > 0.5,
                               jnp.logical_and(m_eq > 0.5,
                                               r1e + starts_e <= quota))
        self_f = sel_m.astype(jnp.float32)

        r1s = jnp.dot(self_f, U128, preferred_element_type=jnp.float32)
        rs = jnp.sum(self_f, axis=1, keepdims=True)
        starts = jnp.dot(Lstrict, rs, preferred_element_type=jnp.float32)
        ends = starts + rs
        starts_r = jnp.dot(ones_1x50, I50 * starts, preferred_element_type=jnp.float32)
        ends_r = jnp.dot(ones_1x50, I50 * ends, preferred_element_type=jnp.float32)
        ohr = jnp.logical_and(kp1 > starts_r, kp1 <= ends_r).astype(jnp.float32)
        rows_rank = jnp.dot(ohr, r1s, preferred_element_type=jnp.float32)
        rows_sel = jnp.dot(ohr, self_f, preferred_element_type=jnp.float32)
        starts_k = jnp.dot(ohr, starts, preferred_element_type=jnp.float32)
        tgt = kp1 - starts_k
        m2 = jnp.logical_and(rows_rank == tgt, rows_sel > 0.5).astype(jnp.float32)
        c_k = jnp.dot(m2, iota128c, preferred_element_type=jnp.float32)
        r_k = jnp.dot(ohr, iota50c, preferred_element_type=jnp.float32)
        r_row = jnp.dot(ones_1x128, I128 * r_k, preferred_element_type=jnp.float32)
        c_row = jnp.dot(ones_1x128, I128 * c_k, preferred_element_type=jnp.float32)
        g_row = r_row * jnp.float32(LANE) + c_row
        selg_ref[b : b + 1, :] = g_row.astype(jnp.int32)


def _select(obj3):
    return pl.pallas_call(
        _select_kernel,
        in_specs=[pl.BlockSpec(memory_space=pltpu.MemorySpace.VMEM)],
        out_specs=pl.BlockSpec(memory_space=pltpu.MemorySpace.VMEM),
        out_shape=jax.ShapeDtypeStruct((BS, LANE), jnp.int32),
    )(obj3)


def _dot_f32(a, b):
    a1 = a.astype(jnp.bfloat16)
    ra = a - a1.astype(jnp.float32)
    a2 = ra.astype(jnp.bfloat16)
    a3 = (ra - a2.astype(jnp.float32)).astype(jnp.bfloat16)
    b1 = b.astype(jnp.bfloat16)
    rb = b - b1.astype(jnp.float32)
    b2 = rb.astype(jnp.bfloat16)
    b3 = (rb - b2.astype(jnp.float32)).astype(jnp.bfloat16)

    def d(x, y):
        return jnp.dot(x, y, preferred_element_type=jnp.float32)

    return ((d(a3, b1) + d(a1, b3)) + (d(a2, b2) +
            (d(a2, b1) + d(a1, b2))) + d(a1, b1))


def _score_kernel(selg_smem, selg_vm, ifeat_hbm, wvst_ref, yf_ref, wts_ref,
                  bts_ref, bvs_ref, max_ref, gb_ref, gbuf, sem):
    copies = []
    for b in range(BS):
        for k in range(SEL):
            idx = selg_smem[b, k] + b * GRID
            cp = pltpu.make_async_copy(
                ifeat_hbm.at[pl.ds(idx, 1), :],
                gbuf.at[pl.ds(b * SEL + k, 1), :],
                sem,
            )
            cp.start()
            copies.append(cp)

    y_new = _dot_f32(yf_ref[...], wts_ref[...]) + bts_ref[...]
    u = _dot_f32(y_new, wvst_ref[...])
    c = jnp.sum(y_new * bvs_ref[...], axis=1, keepdims=True)

    for cp in copies:
        cp.wait()

    g3 = gbuf[...].reshape(BS, SEL, VIS)
    scores = jnp.sum(g3 * u[:, None, :], axis=2) + c
    kio = lax.broadcasted_iota(jnp.int32, (BS, SEL), 1).astype(jnp.float32)
    maxv = jnp.max(scores, axis=1, keepdims=True)
    cand = jnp.where(scores == maxv, kio, jnp.float32(1e9))
    bestk = jnp.min(cand, axis=1, keepdims=True)
    lane = lax.broadcasted_iota(jnp.int32, (BS, LANE), 1).astype(jnp.float32)
    ohk = lane == bestk
    gbest = jnp.sum(jnp.where(ohk, selg_vm[...].astype(jnp.float32), 0.0),
                    axis=1, keepdims=True)
    max_ref[...] = jnp.broadcast_to(maxv, (BS, LANE))
    gb_ref[...] = jnp.broadcast_to(gbest, (BS, LANE)).astype(jnp.int32)


def _score(selg, ifeat2, w_vs_t, y_feat, w_ts, bts2, bvs2):
    return pl.pallas_call(
        _score_kernel,
        in_specs=[
            pl.BlockSpec(memory_space=pltpu.MemorySpace.SMEM),
            pl.BlockSpec(memory_space=pltpu.MemorySpace.VMEM),
            pl.BlockSpec(memory_space=pltpu.MemorySpace.HBM),
            pl.BlockSpec(memory_space=pltpu.MemorySpace.VMEM),
            pl.BlockSpec(memory_space=pltpu.MemorySpace.VMEM),
            pl.BlockSpec(memory_space=pltpu.MemorySpace.VMEM),
            pl.BlockSpec(memory_space=pltpu.MemorySpace.VMEM),
            pl.BlockSpec(memory_space=pltpu.MemorySpace.VMEM),
        ],
        out_specs=[
            pl.BlockSpec(memory_space=pltpu.MemorySpace.VMEM),
            pl.BlockSpec(memory_space=pltpu.MemorySpace.VMEM),
        ],
        out_shape=[
            jax.ShapeDtypeStruct((BS, LANE), jnp.float32),
            jax.ShapeDtypeStruct((BS, LANE), jnp.int32),
        ],
        scratch_shapes=[
            pltpu.VMEM((BS * SEL, VIS), jnp.float32),
            pltpu.SemaphoreType.DMA,
        ],
    )(selg, selg, ifeat2, w_vs_t, y_feat, w_ts, bts2, bvs2)


def _decode_kernel(gb_ref, box_ref, out_ref):
    b = pl.program_id(0)
    sub = jnp.remainder(gb_ref[b], 8)
    ohc = lax.broadcasted_iota(jnp.int32, (8, 1), 0) == sub
    row = jnp.sum(jnp.where(ohc, box_ref[0], 0.0), axis=0, keepdims=True)

    def chan(a, j):
        return row[:, a * CH + j : a * CH + j + 1]

    xs, ys, x1s, y1s, scs = [], [], [], [], []
    for a in range(ANN):
        cx, cy, w, h, sc = (chan(a, 0), chan(a, 1), chan(a, 2), chan(a, 3),
                            chan(a, 4))
        x0 = cx - w * 0.5
        y0 = cy - h * 0.5
        xs.append(x0)
        ys.append(y0)
        x1s.append(x0 + w)
        y1s.append(y0 + h)
        scs.append(sc)

    use0 = jnp.logical_and(scs[0] >= scs[1], scs[0] >= scs[2])
    use1 = jnp.logical_and(jnp.logical_not(use0), scs[1] >= scs[2])

    def pick(v):
        return jnp.where(use0, v[0], jnp.where(use1, v[1], v[2]))

    bx0, by0, bx1, by1, bsc = pick(xs), pick(ys), pick(x1s), pick(y1s), pick(scs)
    lane = lax.broadcasted_iota(jnp.int32, (1, 1, LANE), 2)
    out = jnp.where(lane == 0, bx0,
          jnp.where(lane == 1, by0,
          jnp.where(lane == 2, bx1,
          jnp.where(lane == 3, by1, bsc))))
    out_ref[...] = out


def _decode(gb, boxes3):
    grid_spec = pltpu.PrefetchScalarGridSpec(
        num_scalar_prefetch=1,
        grid=(BS,),
        in_specs=[pl.BlockSpec((1, 8, ROW), lambda b, gb: (b, gb[b] // 8, 0))],
        out_specs=pl.BlockSpec((1, 1, LANE), lambda b, gb: (b, 0, 0)),
    )
    return pl.pallas_call(
        _decode_kernel,
        grid_spec=grid_spec,
        out_shape=jax.ShapeDtypeStruct((BS, 1, LANE), jnp.float32),
    )(gb, boxes3)


@jax.jit
def kernel(boxes_sml0, i_feat, y_feat, W_vs, b_vs, W_ts, b_ts):
    boxes3 = boxes_sml0.reshape(BS, GRID, ROW)
    obj = _objectness(boxes3)
    obj3 = obj.reshape(BS, SUBL, LANE)
    selg = _select(obj3)
    ifeat2 = i_feat.reshape(BS * GRID, VIS)
    maxpad, gbpad = _score(selg, ifeat2, W_vs.T, y_feat, W_ts,
                           b_ts.reshape(1, HID), b_vs.reshape(1, HID))
    maxval = maxpad[:, 0]
    gb = gbpad[:, 0]
    boxrow = _decode(gb, boxes3)
    box_new = boxrow[:, 0, :5].reshape(BS, 1, 5)
    return box_new, maxval
